# Initial kernel scaffold; baseline (speedup 1.0000x reference)
import jax
import jax.numpy as jnp
from jax.experimental import pallas as pl


def kernel(x, emb_table, W, b):
    raise NotImplementedError("write your pallas kernel here")


# speedup vs baseline: 1.4127x; 1.4127x over previous
import functools

import jax
import jax.numpy as jnp
from jax import lax
from jax.experimental import pallas as pl
from jax.experimental.pallas import tpu as pltpu
from jax.experimental.pallas import tpu_sc as plsc

VOCAB = 100000
DIM = 128
NUM_CLASSES = 2
L = 16384

_info = plsc.get_sparse_core_info()
NC, NS, LANES = _info.num_cores, _info.num_subcores, _info.num_lanes
NW = NC * NS
PER_W = L // NW
CHUNK = 128
CHUNKS = PER_W // CHUNK
VPR = DIM // LANES


def _gather_sum_body(xr_hbm, table_hbm, out_hbm, idx_v, rows_v, acc_v, sem):
    wid = lax.axis_index("s") * NC + lax.axis_index("c")
    pltpu.sync_copy(xr_hbm.at[wid], idx_v)
    for j in range(CHUNKS):
        for t in range(CHUNK // LANES):
            v = idx_v[j, pl.ds(t * LANES, LANES)]
            idx_v[j, pl.ds(t * LANES, LANES)] = jnp.where(v == -1, 1, v)

    accs = tuple(jnp.zeros((LANES,), jnp.float32) for _ in range(VPR))
    for j in range(CHUNKS):
        pltpu.async_copy(table_hbm.at[idx_v.at[j]], rows_v, sem).wait()

        def body(r, a):
            return tuple(a[k] + rows_v[r, pl.ds(k * LANES, LANES)]
                         for k in range(VPR))

        accs = lax.fori_loop(0, CHUNK, body, accs)

    for k in range(VPR):
        acc_v[pl.ds(k * LANES, LANES)] = accs[k]
    pltpu.sync_copy(acc_v, out_hbm.at[wid])


_gather_sum = functools.partial(
    pl.kernel,
    out_type=jax.ShapeDtypeStruct((NW, DIM), jnp.float32),
    mesh=plsc.VectorSubcoreMesh(core_axis_name="c", subcore_axis_name="s"),
    scratch_types=[
        pltpu.VMEM((CHUNKS, CHUNK), jnp.int32),
        pltpu.VMEM((CHUNK, DIM), jnp.float32),
        pltpu.VMEM((DIM,), jnp.float32),
        pltpu.SemaphoreType.DMA,
    ],
)(_gather_sum_body)


def _tail_body(p_ref, w_ref, b_ref, o_ref):
    s = jnp.sum(p_ref[...], axis=0, keepdims=True) * (1.0 / L)
    h = jnp.maximum(s, 0.0)
    logits = lax.dot_general(h, w_ref[...], (((1,), (1,)), ((), ())))
    logits = logits + b_ref[...]
    mx = jnp.max(logits, axis=1, keepdims=True)
    lse = mx + jnp.log(jnp.sum(jnp.exp(logits - mx), axis=1, keepdims=True))
    o_ref[...] = logits - lse


_tail = pl.pallas_call(
    _tail_body,
    out_shape=jax.ShapeDtypeStruct((1, NUM_CLASSES), jnp.float32),
)


def kernel(x, emb_table, W, b):
    xr = x.reshape(NW, CHUNKS, CHUNK).astype(jnp.int32)
    partials = _gather_sum(xr, emb_table)
    return _tail(partials, W, b.reshape(1, NUM_CLASSES))


# speedup vs baseline: 1.4966x; 1.0594x over previous
import functools

import jax
import jax.numpy as jnp
from jax import lax
from jax.experimental import pallas as pl
from jax.experimental.pallas import tpu as pltpu
from jax.experimental.pallas import tpu_sc as plsc

VOCAB = 100000
DIM = 128
NUM_CLASSES = 2
L = 16384

_info = plsc.get_sparse_core_info()
NC, NS, LANES = _info.num_cores, _info.num_subcores, _info.num_lanes
NW = NC * NS
PER_W = L // NW
CHUNK = 128
CHUNKS = PER_W // CHUNK
VPR = DIM // LANES


RU = 4


def _gather_sum_body(xr_hbm, table_hbm, out_hbm, idx_v, rows_v, acc_v,
                     sem0, sem1, sem2, sem3):
    sems = (sem0, sem1, sem2, sem3)
    wid = lax.axis_index("s") * NC + lax.axis_index("c")
    pltpu.sync_copy(xr_hbm.at[wid], idx_v)
    for j in range(CHUNKS):
        for t in range(CHUNK // LANES):
            v = idx_v[j, pl.ds(t * LANES, LANES)]
            idx_v[j, pl.ds(t * LANES, LANES)] = jnp.where(v == -1, 1, v)

    cps = [pltpu.async_copy(table_hbm.at[idx_v.at[j]], rows_v.at[j], sems[j])
           for j in range(CHUNKS)]

    accs = tuple(jnp.zeros((LANES,), jnp.float32) for _ in range(VPR))
    for j in range(CHUNKS):
        cps[j].wait()

        def body(r, a, j=j):
            for i in range(RU):
                row = r * RU + i
                a = tuple(a[k] + rows_v[j, row, pl.ds(k * LANES, LANES)]
                          for k in range(VPR))
            return a

        accs = lax.fori_loop(0, CHUNK // RU, body, accs)

    for k in range(VPR):
        acc_v[pl.ds(k * LANES, LANES)] = accs[k]
    pltpu.sync_copy(acc_v, out_hbm.at[wid])


_gather_sum = functools.partial(
    pl.kernel,
    out_type=jax.ShapeDtypeStruct((NW, DIM), jnp.float32),
    mesh=plsc.VectorSubcoreMesh(core_axis_name="c", subcore_axis_name="s"),
    scratch_types=[
        pltpu.VMEM((CHUNKS, CHUNK), jnp.int32),
        pltpu.VMEM((CHUNKS, CHUNK, DIM), jnp.float32),
        pltpu.VMEM((DIM,), jnp.float32),
        pltpu.SemaphoreType.DMA,
        pltpu.SemaphoreType.DMA,
        pltpu.SemaphoreType.DMA,
        pltpu.SemaphoreType.DMA,
    ],
)(_gather_sum_body)


def _tail_body(p_ref, w_ref, b_ref, o_ref):
    s = jnp.sum(p_ref[...], axis=0, keepdims=True) * (1.0 / L)
    h = jnp.maximum(s, 0.0)
    logits = lax.dot_general(h, w_ref[...], (((1,), (1,)), ((), ())))
    logits = logits + b_ref[...]
    mx = jnp.max(logits, axis=1, keepdims=True)
    lse = mx + jnp.log(jnp.sum(jnp.exp(logits - mx), axis=1, keepdims=True))
    o_ref[...] = logits - lse


_tail = pl.pallas_call(
    _tail_body,
    out_shape=jax.ShapeDtypeStruct((1, NUM_CLASSES), jnp.float32),
)


def kernel(x, emb_table, W, b):
    xr = x.reshape(NW, CHUNKS, CHUNK).astype(jnp.int32)
    partials = _gather_sum(xr, emb_table)
    return _tail(partials, W, b.reshape(1, NUM_CLASSES))
